# baseline (speedup 1.0000x reference)
import functools
import jax
import jax.numpy as jnp
from jax import lax
from jax.experimental import pallas as pl
from jax.experimental.pallas import tpu as pltpu
from jax.experimental.pallas import tpu_sc as plsc

NC, NS, L = 2, 16, 16
NW = NC * NS
D = 64
B, S = 4096, 200
N = B * S
V = 1000000
CI = 256
TILES = N // CI
TPW = TILES // NW
W = 256
JOBS = (V - D) // W
JPW = 124
SCALE = 8.0

_mesh = plsc.VectorSubcoreMesh(
    core_axis_name="c", subcore_axis_name="s", num_cores=NC, num_subcores=NS
)
_params = pltpu.CompilerParams(use_tc_tiling_on_sc=True, needs_layout_passes=False)


@functools.partial(
    pl.kernel,
    out_type=jax.ShapeDtypeStruct((V // 2, 2 * D), jnp.float32),
    mesh=_mesh,
    scratch_types=[
        pltpu.VMEM((D, W), jnp.float32),
        pltpu.VMEM((D, W), jnp.float32),
        pltpu.VMEM((W // 2, 2 * D), jnp.float32),
        pltpu.VMEM((W // 2, 2 * D), jnp.float32),
        pltpu.SemaphoreType.DMA,
        pltpu.SemaphoreType.DMA,
        pltpu.SemaphoreType.DMA,
        pltpu.SemaphoreType.DMA,
    ],
    compiler_params=_params,
)
def _prep(tabt_hbm, tailp_hbm, out_hbm, tin0, tin1, tout0, tout1, si0, si1, so0, so1):
    wid = lax.axis_index("s") * NC + lax.axis_index("c")
    iota = lax.iota(jnp.int32, L)
    row_idx = [g * L + iota for g in range(4)]
    tins, touts = (tin0, tin1), (tout0, tout1)
    sis, sos = (si0, si1), (so0, so1)

    def blk_of(t):
        return lax.rem(wid + t * NW, JOBS)

    def in_src(t):
        v0 = pl.multiple_of(blk_of(t) * W, 128)
        return tabt_hbm.at[:, pl.ds(v0, W)]

    def out_dst(t):
        p0 = pl.multiple_of(blk_of(t) * (W // 2), 8)
        return out_hbm.at[pl.ds(p0, W // 2)]

    def transpose_win(tin, tout):
        def prow8(g8, c2):
            sub = tout.at[pl.ds(pl.multiple_of(g8 * 8, 8), 8)]
            cbase = g8 * 16
            for pp in range(8):
                for half in range(2):
                    col = (
                        jnp.broadcast_to(cbase + 2 * pp + half, (L,))
                        .astype(jnp.int32)
                    )
                    for g in range(4):
                        vals = plsc.load_gather(tin, [row_idx[g], col]) * SCALE
                        sub[pp, pl.ds(half * D + g * L, L)] = vals
            return c2

        lax.fori_loop(0, (W // 2) // 8, prow8, 0)

    pltpu.async_copy(in_src(0), tin0, si0)

    def body(u, carry):
        for ph in range(2):
            t = 2 * u + ph
            tin, tout = tins[ph], touts[ph]
            si, so = sis[ph], sos[ph]

            @pl.when(t + 1 < JPW)
            def _():
                pltpu.async_copy(in_src(t + 1), tins[1 - ph], sis[1 - ph])

            pltpu.make_async_copy(in_src(t), tin, si).wait()

            @pl.when(t >= 2)
            def _():
                pltpu.make_async_copy(tout, out_dst(t), so).wait()

            transpose_win(tin, tout)
            pltpu.async_copy(tout, out_dst(t), so)
        return carry

    lax.fori_loop(0, JPW // 2, body, 0)
    pltpu.make_async_copy(tout0, out_dst(0), so0).wait()
    pltpu.make_async_copy(tout1, out_dst(1), so1).wait()

    @pl.when(wid == 0)
    def _():
        pltpu.sync_copy(tailp_hbm, tout0.at[pl.ds(0, 32)])
        pltpu.sync_copy(tout0.at[pl.ds(0, 32)], out_hbm.at[pl.ds(JOBS * (W // 2), 32)])


@functools.partial(
    pl.kernel,
    out_type=jax.ShapeDtypeStruct((S * D, B), jnp.float32),
    mesh=_mesh,
    scratch_types=[
        pltpu.VMEM((TPW * CI,), jnp.int32),
        pltpu.VMEM((CI,), jnp.int32),
        pltpu.VMEM((CI,), jnp.int32),
        pltpu.VMEM((CI, 2 * D), jnp.float32),
        pltpu.VMEM((CI, 2 * D), jnp.float32),
        pltpu.VMEM((D, CI), jnp.float32),
        pltpu.VMEM((D, CI), jnp.float32),
        pltpu.SemaphoreType.DMA,
        pltpu.SemaphoreType.DMA,
        pltpu.SemaphoreType.DMA,
        pltpu.SemaphoreType.DMA,
    ],
    compiler_params=_params,
)
def _emb(
    xf_hbm, tab2_hbm, out_hbm, idxall, pidx0, pidx1,
    rg0, rg1, ob0, ob1, sg0, sg1, so0, so1,
):
    wid = lax.axis_index("s") * NC + lax.axis_index("c")
    iota = lax.iota(jnp.int32, L)
    base = wid * TPW
    pidxs, rgs, obs = (pidx0, pidx1), (rg0, rg1), (ob0, ob1)
    sgs, sos = (sg0, sg1), (so0, so1)

    pltpu.sync_copy(
        xf_hbm.at[pl.ds(pl.multiple_of(base * CI, 8), TPW * CI)], idxall
    )

    def mk_pidx(t, dst):
        def g16(g, c2):
            sl = pl.ds(g * L, L)
            dst[sl] = lax.shift_right_logical(idxall[pl.ds(t * CI + g * L, L)], 1)
            return c2

        lax.fori_loop(0, CI // L, g16, 0)

    def out_dst(t):
        tid = base + t
        j = lax.shift_right_logical(tid, 4)
        ic = tid & 15
        return out_hbm.at[
            pl.ds(pl.multiple_of(j * D, 8), D),
            pl.ds(pl.multiple_of(ic * CI, 128), CI),
        ]

    def transpose_tile(t, rg, ob):
        def kgrp(k16, c2):
            rvec = k16 * L + iota
            hvec = (idxall[pl.ds(t * CI + k16 * L, L)] & 1) * D
            dst = pl.ds(k16 * L, L)
            for d8 in range(D // 8):
                sub = ob.at[pl.ds(d8 * 8, 8)]
                for dd in range(8):
                    vals = plsc.load_gather(rg, [rvec, hvec + (d8 * 8 + dd)])
                    sub[dd, dst] = vals
            return c2

        lax.fori_loop(0, CI // L, kgrp, 0)

    mk_pidx(0, pidx0)
    pltpu.async_copy(tab2_hbm.at[pidx0], rg0, sg0)

    def body(u, carry):
        for ph in range(2):
            t = 2 * u + ph
            rg, ob = rgs[ph], obs[ph]
            sg, so = sgs[ph], sos[ph]

            @pl.when(t + 1 < TPW)
            def _():
                mk_pidx(t + 1, pidxs[1 - ph])
                pltpu.async_copy(tab2_hbm.at[pidxs[1 - ph]], rgs[1 - ph], sgs[1 - ph])

            pltpu.make_async_copy(tab2_hbm.at[pidxs[ph]], rg, sg).wait()

            @pl.when(t >= 2)
            def _():
                pltpu.make_async_copy(ob, out_dst(t), so).wait()

            transpose_tile(t, rg, ob)
            pltpu.async_copy(ob, out_dst(t), so)
        return carry

    lax.fori_loop(0, TPW // 2, body, 0)
    pltpu.make_async_copy(ob0, out_dst(0), so0).wait()
    pltpu.make_async_copy(ob1, out_dst(1), so1).wait()


def kernel(x, table):
    xf = x.T.reshape(-1)
    tailp = (lax.slice(table, (JOBS * W, 0), (V, D)) * SCALE).reshape(32, 128)
    tab2 = _prep(table.T, tailp)
    out = _emb(xf, tab2)
    return out.reshape(S, D, B).transpose(2, 0, 1)


# speedup vs baseline: 1.9519x; 1.9519x over previous
import functools
import jax
import jax.numpy as jnp
from jax import lax
from jax.experimental import pallas as pl
from jax.experimental.pallas import tpu as pltpu
from jax.experimental.pallas import tpu_sc as plsc

NC, NS = 2, 16
NW = NC * NS
D = 64
B, S = 4096, 200
N = B * S
CI = 256
TILES = N // CI
TPW = TILES // NW
SCALE = 8.0

_mesh = plsc.VectorSubcoreMesh(
    core_axis_name="c", subcore_axis_name="s", num_cores=NC, num_subcores=NS
)


@functools.partial(
    pl.kernel,
    out_type=jax.ShapeDtypeStruct((N, D), jnp.float32),
    mesh=_mesh,
    scratch_types=[
        pltpu.VMEM((CI,), jnp.int32),
        pltpu.VMEM((CI,), jnp.int32),
        pltpu.VMEM((CI, D), jnp.float32),
        pltpu.VMEM((CI, D), jnp.float32),
        pltpu.SemaphoreType.DMA,
        pltpu.SemaphoreType.DMA,
        pltpu.SemaphoreType.DMA,
        pltpu.SemaphoreType.DMA,
    ],
    compiler_params=pltpu.CompilerParams(
        use_tc_tiling_on_sc=False, needs_layout_passes=False
    ),
)
def _gather(xf_hbm, tab_hbm, out_hbm, i0, i1, r0, r1, sg0, sg1, so0, so1):
    wid = lax.axis_index("s") * NC + lax.axis_index("c")
    idxs, rgs = (i0, i1), (r0, r1)
    sgs, sos = (sg0, sg1), (so0, so1)

    def toff(t):
        return pl.multiple_of((wid + t * NW) * CI, 8)

    def start(t, ph):
        pltpu.sync_copy(xf_hbm.at[pl.ds(toff(t), CI)], idxs[ph])
        pltpu.async_copy(tab_hbm.at[idxs[ph]], rgs[ph], sgs[ph])

    start(0, 0)

    def body(u, carry):
        for ph in range(2):
            t = 2 * u + ph

            @pl.when(t >= 1)
            def _():
                pltpu.make_async_copy(
                    rgs[1 - ph], out_hbm.at[pl.ds(toff(t - 1), CI)], sos[1 - ph]
                ).wait()

            @pl.when(t + 1 < TPW)
            def _():
                start(t + 1, 1 - ph)

            pltpu.make_async_copy(tab_hbm.at[idxs[ph]], rgs[ph], sgs[ph]).wait()
            pltpu.async_copy(rgs[ph], out_hbm.at[pl.ds(toff(t), CI)], sos[ph])
        return carry

    lax.fori_loop(0, TPW // 2, body, 0)
    pltpu.make_async_copy(
        rgs[1], out_hbm.at[pl.ds(toff(TPW - 1), CI)], sos[1]
    ).wait()


def kernel(x, table):
    xf = x.T.reshape(-1)
    out = _gather(xf, table)
    return out.reshape(S, B, D).transpose(1, 0, 2) * SCALE
